# baseline (speedup 1.0000x reference)
import functools

import numpy as np

import jax
import jax.numpy as jnp
from jax import lax
from jax.experimental import pallas as pl
from jax.experimental.pallas import tpu as pltpu
from jax.experimental.pallas import tpu_sc as plsc

N_TOK = 204800
N_BAGS = 4096
VOCAB = 100000
EMB_D = 64
DW = EMB_D // 2
NQ = EMB_D // 16
NC = 2
NS = 16
NW = NC * NS
BPW = N_BAGS // NW
C = 512
CSUB = 4
CB = C // CSUB

def _tile_body(in_hbm, off_hbm, w_hbm, tab_hbm, out_hbm,
               offs_v, idx_v, w_v, rows_v, acc_v,
               lin0, lin1, lin2, gat0, gat1, gat2):
    lin_sem = (lin0, lin1, lin2)
    gat_sem = (gat0, gat1, gat2)
    wid = lax.axis_index("s") * NC + lax.axis_index("c")
    base_bag = wid * BPW

    pltpu.sync_copy(off_hbm.at[pl.ds(base_bag, BPW)], offs_v.at[pl.ds(0, BPW)])

    @pl.when(wid == NW - 1)
    def _():
        offs_v[pl.ds(BPW, 16)] = jnp.full((16,), N_TOK, jnp.int32)

    @pl.when(wid < NW - 1)
    def _():
        pltpu.sync_copy(off_hbm.at[pl.ds(base_bag + BPW, 16)],
                        offs_v.at[pl.ds(BPW, 16)])

    def zero_body(kz, carry):
        for q in range(NQ):
            acc_v[kz, pl.ds(16 * q, 16)] = jnp.zeros((16,), jnp.float32)
        return carry

    lax.fori_loop(0, BPW, zero_body, 0)

    start = offs_v[pl.ds(0, 16)][0]
    end = offs_v[pl.ds(BPW, 16)][0]
    c0 = start // C
    c1 = (end + (C - 1)) // C

    def fire_lin(c, b):
        for i in range(CSUB):
            pltpu.async_copy(in_hbm.at[pl.ds(c * C + CB * i, CB)],
                             idx_v.at[b, i], lin_sem[b])
        pltpu.async_copy(w_hbm.at[pl.ds(c * C, C)],
                         w_v.at[b, pl.ds(0, C)], lin_sem[b])

    def wait_lin(b):
        for i in range(CSUB):
            pltpu.make_async_copy(in_hbm.at[pl.ds(CB * i, CB)],
                                  idx_v.at[b, i], lin_sem[b]).wait()
        pltpu.make_async_copy(w_hbm.at[pl.ds(0, C)],
                              w_v.at[b, pl.ds(0, C)], lin_sem[b]).wait()

    def fire_gat(b):
        for i in range(CSUB):
            pltpu.async_copy(tab_hbm.at[idx_v.at[b, i]],
                             rows_v.at[b, pl.ds(CB * i, CB)], gat_sem[b])

    def wait_gat(b):
        for i in range(CSUB):
            pltpu.make_async_copy(tab_hbm.at[idx_v.at[b, i]],
                                  rows_v.at[b, pl.ds(CB * i, CB)],
                                  gat_sem[b]).wait()

    @pl.when(c0 < c1)
    def _():
        fire_lin(c0, 0)

    @pl.when(c0 + 1 < c1)
    def _():
        fire_lin(c0 + 1, 1)

    @pl.when(c0 + 2 < c1)
    def _():
        fire_lin(c0 + 2, 2)

    @pl.when(c0 < c1)
    def _():
        wait_lin(0)
        fire_gat(0)

    @pl.when(c0 + 1 < c1)
    def _():
        wait_lin(1)
        fire_gat(1)

    def compute_chunk(c, b, k):
        cbase = c * C
        hi = jnp.minimum(end, cbase + C)

        def count_body(g, cnt):
            d = hi - offs_v[pl.ds(16 * g, 16)]
            m = jnp.minimum(jnp.maximum(d, 0), 1)
            return cnt + jnp.sum(m)

        n_lt = lax.fori_loop(0, BPW // 16, count_body, jnp.int32(0))

        def bag_body(kk, carry):
            o01 = offs_v[pl.ds(kk, 16)]
            lo = jnp.maximum(o01[0], cbase)
            hi_k = jnp.minimum(o01[1], hi)

            def tok_body(p, accs):
                j = p - cbase
                w = w_v[b, pl.ds(j, 16)][0]
                return tuple(accs[q] + w * rows_v[b, j, pl.ds(16 * q, 16)]
                             for q in range(NQ))

            accs = lax.fori_loop(
                lo, hi_k, tok_body,
                tuple(jnp.zeros((16,), jnp.float32) for _ in range(NQ)))

            @pl.when(hi_k > lo)
            def _():
                for q in range(NQ):
                    plsc.addupdate(acc_v.at[kk, pl.ds(16 * q, 16)], accs[q])

            return carry

        lax.fori_loop(k, n_lt, bag_body, jnp.int32(0))

        straddle = offs_v[pl.ds(n_lt, 16)][0] > hi
        return jnp.maximum(k, n_lt - jnp.where(straddle, 1, 0))

    def triple_body(i3, k):
        for u in range(3):
            c = c0 + 3 * i3 + u
            b = u
            b2 = (u + 2) % 3

            @pl.when(c + 2 < c1)
            def _():
                wait_lin(b2)
                fire_gat(b2)

            @pl.when(c < c1)
            def _():
                wait_gat(b)

            k = compute_chunk(c, b, k)

            @pl.when(c + 3 < c1)
            def _():
                fire_lin(c + 3, b)
        return k

    nthird = (c1 - c0 + 2) // 3
    lax.fori_loop(0, nthird, triple_body, jnp.int32(0))

    pltpu.sync_copy(acc_v, out_hbm.at[pl.ds(base_bag, BPW)])


def kernel(input_, offsets, emb_weights, emb_table):
    mesh = plsc.VectorSubcoreMesh(core_axis_name="c", subcore_axis_name="s",
                                  num_cores=NC, num_subcores=NS)
    run = pl.kernel(
        _tile_body,
        out_type=jax.ShapeDtypeStruct((N_BAGS, EMB_D), jnp.float32),
        mesh=mesh,
        scratch_types=[
            pltpu.VMEM((BPW + 16,), jnp.int32),
            pltpu.VMEM((3, CSUB, CB), jnp.int32),
            pltpu.VMEM((3, C + 16), jnp.float32),
            pltpu.VMEM((3, C, EMB_D), jnp.float32),
            pltpu.VMEM((BPW, EMB_D), jnp.float32),
            pltpu.SemaphoreType.DMA,
            pltpu.SemaphoreType.DMA,
            pltpu.SemaphoreType.DMA,
            pltpu.SemaphoreType.DMA,
            pltpu.SemaphoreType.DMA,
            pltpu.SemaphoreType.DMA,
        ],
        compiler_params=pltpu.CompilerParams(needs_layout_passes=False,
                                             use_tc_tiling_on_sc=False),
    )
    return run(input_.astype(jnp.int32), offsets.astype(jnp.int32),
               emb_weights, emb_table)


# speedup vs baseline: 1.0090x; 1.0090x over previous
import functools

import numpy as np

import jax
import jax.numpy as jnp
from jax import lax
from jax.experimental import pallas as pl
from jax.experimental.pallas import tpu as pltpu
from jax.experimental.pallas import tpu_sc as plsc

N_TOK = 204800
N_BAGS = 4096
VOCAB = 100000
EMB_D = 64
DW = EMB_D // 2
NQ = EMB_D // 16
NC = 2
NS = 16
NW = NC * NS
BPW = N_BAGS // NW
C = 512
CSUB = 4
CB = C // CSUB

def _tile_body(in_hbm, off_hbm, w_hbm, tab_hbm, out_hbm,
               offs_v, idx_v, w_v, rows_v, acc_v,
               lin0, lin1, gat0, gat1):
    lin_sem = (lin0, lin1)
    gat_sem = (gat0, gat1)
    wid = lax.axis_index("s") * NC + lax.axis_index("c")
    base_bag = wid * BPW

    pltpu.sync_copy(off_hbm.at[pl.ds(base_bag, BPW)], offs_v.at[pl.ds(0, BPW)])

    @pl.when(wid == NW - 1)
    def _():
        offs_v[pl.ds(BPW, 16)] = jnp.full((16,), N_TOK, jnp.int32)

    @pl.when(wid < NW - 1)
    def _():
        pltpu.sync_copy(off_hbm.at[pl.ds(base_bag + BPW, 16)],
                        offs_v.at[pl.ds(BPW, 16)])

    def zero_body(kz, carry):
        for q in range(NQ):
            acc_v[kz, pl.ds(16 * q, 16)] = jnp.zeros((16,), jnp.float32)
        return carry

    lax.fori_loop(0, BPW, zero_body, 0)

    start = offs_v[pl.ds(0, 16)][0]
    end = offs_v[pl.ds(BPW, 16)][0]
    c0 = start // C
    c1 = (end + (C - 1)) // C

    def fire_lin(c, b):
        for i in range(CSUB):
            pltpu.async_copy(in_hbm.at[pl.ds(c * C + CB * i, CB)],
                             idx_v.at[b, i], lin_sem[b])
        pltpu.async_copy(w_hbm.at[pl.ds(c * C, C)],
                         w_v.at[b, pl.ds(0, C)], lin_sem[b])

    def wait_lin(b):
        for i in range(CSUB):
            pltpu.make_async_copy(in_hbm.at[pl.ds(CB * i, CB)],
                                  idx_v.at[b, i], lin_sem[b]).wait()
        pltpu.make_async_copy(w_hbm.at[pl.ds(0, C)],
                              w_v.at[b, pl.ds(0, C)], lin_sem[b]).wait()

    def fire_gat(b):
        for i in range(CSUB):
            pltpu.async_copy(tab_hbm.at[idx_v.at[b, i]],
                             rows_v.at[b, pl.ds(CB * i, CB)], gat_sem[b])

    def wait_gat(b):
        for i in range(CSUB):
            pltpu.make_async_copy(tab_hbm.at[idx_v.at[b, i]],
                                  rows_v.at[b, pl.ds(CB * i, CB)],
                                  gat_sem[b]).wait()

    @pl.when(c0 < c1)
    def _():
        fire_lin(c0, 0)

    @pl.when(c0 + 1 < c1)
    def _():
        fire_lin(c0 + 1, 1)

    @pl.when(c0 < c1)
    def _():
        wait_lin(0)
        fire_gat(0)

    def compute_chunk(c, b, k):
        cbase = c * C
        hi = jnp.minimum(end, cbase + C)

        def count_body(g, cnt):
            d = hi - offs_v[pl.ds(16 * g, 16)]
            m = jnp.minimum(jnp.maximum(d, 0), 1)
            return cnt + jnp.sum(m)

        n_lt = lax.fori_loop(0, BPW // 16, count_body, jnp.int32(0))

        def bag_body(kk, carry):
            o01 = offs_v[pl.ds(kk, 16)]
            lo = jnp.maximum(o01[0], cbase)
            hi_k = jnp.minimum(o01[1], hi)

            def tok_body(p, accs):
                j = p - cbase
                w = w_v[b, pl.ds(j, 16)][0]
                return tuple(accs[q] + w * rows_v[b, j, pl.ds(16 * q, 16)]
                             for q in range(NQ))

            accs = lax.fori_loop(
                lo, hi_k, tok_body,
                tuple(jnp.zeros((16,), jnp.float32) for _ in range(NQ)))

            @pl.when(hi_k > lo)
            def _():
                for q in range(NQ):
                    plsc.addupdate(acc_v.at[kk, pl.ds(16 * q, 16)], accs[q])

            return carry

        lax.fori_loop(k, n_lt, bag_body, jnp.int32(0))

        straddle = offs_v[pl.ds(n_lt, 16)][0] > hi
        return jnp.maximum(k, n_lt - jnp.where(straddle, 1, 0))

    def pair_body(i2, k):
        for u in range(2):
            c = c0 + 2 * i2 + u
            b = u

            @pl.when(c + 1 < c1)
            def _():
                wait_lin(1 - b)
                fire_gat(1 - b)

            @pl.when(c < c1)
            def _():
                wait_gat(b)

            k = compute_chunk(c, b, k)

            @pl.when(c + 2 < c1)
            def _():
                fire_lin(c + 2, b)
        return k

    nhalf = (c1 - c0 + 1) // 2
    lax.fori_loop(0, nhalf, pair_body, jnp.int32(0))

    pltpu.sync_copy(acc_v, out_hbm.at[pl.ds(base_bag, BPW)])


def kernel(input_, offsets, emb_weights, emb_table):
    mesh = plsc.VectorSubcoreMesh(core_axis_name="c", subcore_axis_name="s",
                                  num_cores=NC, num_subcores=NS)
    run = pl.kernel(
        _tile_body,
        out_type=jax.ShapeDtypeStruct((N_BAGS, EMB_D), jnp.float32),
        mesh=mesh,
        scratch_types=[
            pltpu.VMEM((BPW + 16,), jnp.int32),
            pltpu.VMEM((2, CSUB, CB), jnp.int32),
            pltpu.VMEM((2, C + 16), jnp.float32),
            pltpu.VMEM((2, C, EMB_D), jnp.float32),
            pltpu.VMEM((BPW, EMB_D), jnp.float32),
            pltpu.SemaphoreType.DMA,
            pltpu.SemaphoreType.DMA,
            pltpu.SemaphoreType.DMA,
            pltpu.SemaphoreType.DMA,
        ],
        compiler_params=pltpu.CompilerParams(needs_layout_passes=False,
                                             use_tc_tiling_on_sc=False),
    )
    return run(input_.astype(jnp.int32), offsets.astype(jnp.int32),
               emb_weights, emb_table)
